# Initial kernel scaffold; baseline (speedup 1.0000x reference)
import functools

import jax
import jax.numpy as jnp
from jax import lax
from jax.experimental import pallas as pl
from jax.experimental.pallas import tpu as pltpu
from jax.experimental.pallas import tpu_sc as plsc

_N = 10000
_E = 320000
_H = 128
_C = 40
_NC = 2
_NS = 16
_NW = _NC * _NS
_EPT = _E // _NW
_CHUNK = 128
_NFULL = _EPT // _CHUNK
_TAIL = _EPT - _NFULL * _CHUNK
_RPT = _N // _NS
_ZR = 125
_ROWB = 2000

_sc_mesh = plsc.VectorSubcoreMesh(core_axis_name="c", subcore_axis_name="s")


@functools.partial(
    pl.kernel,
    out_type=jax.ShapeDtypeStruct((_NC, _N, 16), jnp.float32),
    mesh=_sc_mesh,
    scratch_types=[
        pltpu.VMEM((_CHUNK,), jnp.int32),
        pltpu.VMEM((_TAIL,), jnp.int32),
        pltpu.VMEM((_CHUNK, 16), jnp.float32),
        pltpu.VMEM((_RPT, 16), jnp.float32),
        pltpu.VMEM_SHARED((_N, 16), jnp.float32),
        pltpu.SemaphoreType.DMA,
    ],
)
def _deg_call(ei_hbm, out_hbm, dst_v, dst_t, ones_v, zbuf, acc_sh, sem):
    c = lax.axis_index("c")
    s = lax.axis_index("s")
    wid = c * _NS + s

    @pl.loop(0, _RPT)
    def _(i):
        zbuf[i, :] = jnp.zeros((16,), jnp.float32)

    @pl.loop(0, _CHUNK)
    def _(i):
        ones_v[i, :] = jnp.full((16,), 1.0, jnp.float32)

    pltpu.sync_copy(zbuf, acc_sh.at[pl.ds(s * _RPT, _RPT), :])
    plsc.subcore_barrier()

    base0 = wid * _EPT

    @pl.loop(0, _NFULL)
    def _(i):
        base = base0 + i * _CHUNK
        pltpu.sync_copy(ei_hbm.at[1, pl.ds(base, _CHUNK)], dst_v)
        pltpu.sync_copy(ones_v, acc_sh.at[dst_v], add=True)

    tbase = base0 + _NFULL * _CHUNK
    pltpu.sync_copy(ei_hbm.at[1, pl.ds(tbase, _TAIL)], dst_t)
    pltpu.sync_copy(ones_v.at[pl.ds(0, _TAIL), :], acc_sh.at[dst_t], add=True)

    plsc.subcore_barrier()
    pltpu.sync_copy(acc_sh.at[pl.ds(s * _RPT, _RPT), :],
                    out_hbm.at[c, pl.ds(s * _RPT, _RPT), :])


@functools.partial(
    pl.kernel,
    out_type=jax.ShapeDtypeStruct((_NC, _N, _H), jnp.float32),
    mesh=_sc_mesh,
    scratch_types=[
        pltpu.VMEM((_CHUNK,), jnp.int32),
        pltpu.VMEM((_CHUNK,), jnp.int32),
        pltpu.VMEM((_CHUNK, _H), jnp.float32),
        pltpu.VMEM((_TAIL,), jnp.int32),
        pltpu.VMEM((_TAIL,), jnp.int32),
        pltpu.VMEM((_TAIL, _H), jnp.float32),
        pltpu.VMEM((_ZR, _H), jnp.float32),
        pltpu.VMEM_SHARED((_N, _H), jnp.float32),
        pltpu.SemaphoreType.DMA,
    ],
)
def _agg_call(hp_hbm, ei_hbm, out_hbm, src_v, dst_v, rows_v,
              src_t, dst_t, rows_t, zbuf, acc_sh, sem):
    c = lax.axis_index("c")
    s = lax.axis_index("s")
    wid = c * _NS + s

    @pl.loop(0, _ZR)
    def _(i):
        for j in range(_H // 16):
            zbuf[i, pl.ds(j * 16, 16)] = jnp.zeros((16,), jnp.float32)

    for k in range(_RPT // _ZR):
        pltpu.sync_copy(zbuf, acc_sh.at[pl.ds(s * _RPT + k * _ZR, _ZR), :])
    plsc.subcore_barrier()

    base0 = wid * _EPT

    @pl.loop(0, _NFULL)
    def _(i):
        base = base0 + i * _CHUNK
        pltpu.sync_copy(ei_hbm.at[0, pl.ds(base, _CHUNK)], src_v)
        pltpu.sync_copy(ei_hbm.at[1, pl.ds(base, _CHUNK)], dst_v)
        pltpu.async_copy(hp_hbm.at[src_v], rows_v, sem).wait()
        pltpu.sync_copy(rows_v, acc_sh.at[dst_v], add=True)

    tbase = base0 + _NFULL * _CHUNK
    pltpu.sync_copy(ei_hbm.at[0, pl.ds(tbase, _TAIL)], src_t)
    pltpu.sync_copy(ei_hbm.at[1, pl.ds(tbase, _TAIL)], dst_t)
    pltpu.async_copy(hp_hbm.at[src_t], rows_t, sem).wait()
    pltpu.sync_copy(rows_t, acc_sh.at[dst_t], add=True)

    plsc.subcore_barrier()
    pltpu.sync_copy(acc_sh.at[pl.ds(s * _RPT, _RPT), :],
                    out_hbm.at[c, pl.ds(s * _RPT, _RPT), :])


_PREC = lax.Precision.HIGHEST


def _mm1_body(p0_ref, p1_ref, x_ref, w_ref, oh_ref, od_ref):
    deg = 1.0 + p0_ref[...] + p1_ref[...]
    dinv = lax.rsqrt(deg)
    g = jnp.dot(x_ref[...], w_ref[...],
                preferred_element_type=jnp.float32, precision=_PREC)
    oh_ref[...] = g * dinv
    od_ref[...] = dinv


@jax.jit
def _mm1_call(p0, p1, x, w):
    grid = (_N // _ROWB,)
    return pl.pallas_call(
        _mm1_body,
        grid=grid,
        in_specs=[
            pl.BlockSpec((_ROWB, 1), lambda i: (i, 0)),
            pl.BlockSpec((_ROWB, 1), lambda i: (i, 0)),
            pl.BlockSpec((_ROWB, _H), lambda i: (i, 0)),
            pl.BlockSpec((_H, _H), lambda i: (0, 0)),
        ],
        out_specs=[
            pl.BlockSpec((_ROWB, _H), lambda i: (i, 0)),
            pl.BlockSpec((_ROWB, 1), lambda i: (i, 0)),
        ],
        out_shape=[
            jax.ShapeDtypeStruct((_N, _H), jnp.float32),
            jax.ShapeDtypeStruct((_N, 1), jnp.float32),
        ],
    )(p0, p1, x, w)


def _layer_body(s0_ref, s1_ref, hp_ref, d_ref, b_ref, w_ref, o_ref):
    y = d_ref[...] * (s0_ref[...] + s1_ref[...] + hp_ref[...]) + b_ref[...]
    y = jnp.maximum(y, 0.0)
    o_ref[...] = jnp.dot(y, w_ref[...],
                         preferred_element_type=jnp.float32,
                         precision=_PREC) * d_ref[...]


@jax.jit
def _layer_call(s0, s1, hp, dinv, b, w):
    grid = (_N // _ROWB,)
    return pl.pallas_call(
        _layer_body,
        grid=grid,
        in_specs=[
            pl.BlockSpec((_ROWB, _H), lambda i: (i, 0)),
            pl.BlockSpec((_ROWB, _H), lambda i: (i, 0)),
            pl.BlockSpec((_ROWB, _H), lambda i: (i, 0)),
            pl.BlockSpec((_ROWB, 1), lambda i: (i, 0)),
            pl.BlockSpec((1, _H), lambda i: (0, 0)),
            pl.BlockSpec((_H, _H), lambda i: (0, 0)),
        ],
        out_specs=pl.BlockSpec((_ROWB, _H), lambda i: (i, 0)),
        out_shape=jax.ShapeDtypeStruct((_N, _H), jnp.float32),
    )(s0, s1, hp, dinv, b, w)


def _final_body(s0_ref, s1_ref, hp_ref, d_ref, b_ref, wl_ref, bl_ref, o_ref):
    y = d_ref[...] * (s0_ref[...] + s1_ref[...] + hp_ref[...]) + b_ref[...]
    y = jnp.maximum(y, 0.0)
    o_ref[...] = jnp.dot(y, wl_ref[...],
                         preferred_element_type=jnp.float32,
                         precision=_PREC) + bl_ref[...]


@jax.jit
def _final_call(s0, s1, hp, dinv, b, wl, bl):
    grid = (_N // _ROWB,)
    return pl.pallas_call(
        _final_body,
        grid=grid,
        in_specs=[
            pl.BlockSpec((_ROWB, _H), lambda i: (i, 0)),
            pl.BlockSpec((_ROWB, _H), lambda i: (i, 0)),
            pl.BlockSpec((_ROWB, _H), lambda i: (i, 0)),
            pl.BlockSpec((_ROWB, 1), lambda i: (i, 0)),
            pl.BlockSpec((1, _H), lambda i: (0, 0)),
            pl.BlockSpec((_H, _C), lambda i: (0, 0)),
            pl.BlockSpec((1, _C), lambda i: (0, 0)),
        ],
        out_specs=pl.BlockSpec((_ROWB, _C), lambda i: (i, 0)),
        out_shape=jax.ShapeDtypeStruct((_N, _C), jnp.float32),
    )(s0, s1, hp, dinv, b, wl, bl)


@jax.jit
def kernel(x, edge_index, W1, b1, W2, b2, W3, b3, Wl, bl):
    ei = edge_index
    degp = _deg_call(ei)
    h1 = None
    p0 = degp[0, :, :1]
    p1 = degp[1, :, :1]
    h1, dinv = _mm1_call(p0, p1, x, W1)
    s = _agg_call(h1, ei)
    h2 = _layer_call(s[0], s[1], h1, dinv, b1.reshape(1, _H), W2)
    s = _agg_call(h2, ei)
    h3 = _layer_call(s[0], s[1], h2, dinv, b2.reshape(1, _H), W3)
    s = _agg_call(h3, ei)
    out = _final_call(s[0], s[1], h3, dinv, b3.reshape(1, _H), Wl,
                      bl.reshape(1, _C))
    return out


# speedup vs baseline: 13.0156x; 13.0156x over previous
import functools

import jax
import jax.numpy as jnp
from jax import lax
from jax.experimental import pallas as pl
from jax.experimental.pallas import tpu as pltpu
from jax.experimental.pallas import tpu_sc as plsc

_N = 10000
_E = 320000
_H = 128
_C = 40
_NC = 2
_NS = 16
_NW = _NC * _NS
_EPT = _E // _NW
_CHUNK = 128
_NFULL = _EPT // _CHUNK
_TAIL = _EPT - _NFULL * _CHUNK
_NP = 10240
_RPT = _NP // _NS
_ZR = 128
_ROWB = 2000

_sc_mesh = plsc.VectorSubcoreMesh(core_axis_name="c", subcore_axis_name="s")


@functools.partial(
    pl.kernel,
    out_type=jax.ShapeDtypeStruct((_NC, _NP, _H), jnp.float32),
    mesh=_sc_mesh,
    scratch_types=[
        pltpu.VMEM((_CHUNK,), jnp.int32),
        pltpu.VMEM((_TAIL,), jnp.int32),
        pltpu.VMEM((_CHUNK, _H), jnp.float32),
        pltpu.VMEM((_ZR, _H), jnp.float32),
        pltpu.VMEM_SHARED((_NP, _H), jnp.float32),
        pltpu.SemaphoreType.DMA,
    ],
)
def _deg_call(dst_hbm, out_hbm, dst_v, dst_t, ones_v, zbuf, acc_sh, sem):
    c = lax.axis_index("c")
    s = lax.axis_index("s")
    wid = c * _NS + s

    @pl.loop(0, _ZR)
    def _(i):
        for j in range(_H // 16):
            zbuf[i, pl.ds(j * 16, 16)] = jnp.zeros((16,), jnp.float32)

    @pl.loop(0, _CHUNK)
    def _(i):
        for j in range(_H // 16):
            ones_v[i, pl.ds(j * 16, 16)] = jnp.full((16,), 1.0, jnp.float32)

    for k in range(_RPT // _ZR):
        pltpu.sync_copy(zbuf, acc_sh.at[pl.ds(s * _RPT + k * _ZR, _ZR), :])
    plsc.subcore_barrier()

    base0 = wid * _EPT

    @pl.loop(0, _NFULL)
    def _(i):
        base = base0 + i * _CHUNK
        pltpu.sync_copy(dst_hbm.at[pl.ds(base, _CHUNK)], dst_v)
        pltpu.sync_copy(ones_v, acc_sh.at[dst_v], add=True)

    tbase = base0 + _NFULL * _CHUNK
    pltpu.sync_copy(dst_hbm.at[pl.ds(tbase, _TAIL)], dst_t)
    pltpu.sync_copy(ones_v.at[pl.ds(0, _TAIL), :], acc_sh.at[dst_t], add=True)

    plsc.subcore_barrier()
    pltpu.sync_copy(acc_sh.at[pl.ds(s * _RPT, _RPT), :],
                    out_hbm.at[c, pl.ds(s * _RPT, _RPT), :])


@functools.partial(
    pl.kernel,
    out_type=jax.ShapeDtypeStruct((_NC, _NP, _H), jnp.float32),
    mesh=_sc_mesh,
    scratch_types=[
        pltpu.VMEM((_CHUNK,), jnp.int32),
        pltpu.VMEM((_CHUNK,), jnp.int32),
        pltpu.VMEM((_CHUNK, _H), jnp.float32),
        pltpu.VMEM((_TAIL,), jnp.int32),
        pltpu.VMEM((_TAIL,), jnp.int32),
        pltpu.VMEM((_TAIL, _H), jnp.float32),
        pltpu.VMEM((_ZR, _H), jnp.float32),
        pltpu.VMEM_SHARED((_NP, _H), jnp.float32),
        pltpu.SemaphoreType.DMA,
    ],
)
def _agg_call(hp_hbm, src_hbm, dst_hbm, out_hbm, src_v, dst_v, rows_v,
              src_t, dst_t, rows_t, zbuf, acc_sh, sem):
    c = lax.axis_index("c")
    s = lax.axis_index("s")
    wid = c * _NS + s

    @pl.loop(0, _ZR)
    def _(i):
        for j in range(_H // 16):
            zbuf[i, pl.ds(j * 16, 16)] = jnp.zeros((16,), jnp.float32)

    for k in range(_RPT // _ZR):
        pltpu.sync_copy(zbuf, acc_sh.at[pl.ds(s * _RPT + k * _ZR, _ZR), :])
    plsc.subcore_barrier()

    base0 = wid * _EPT

    @pl.loop(0, _NFULL)
    def _(i):
        base = base0 + i * _CHUNK
        pltpu.sync_copy(src_hbm.at[pl.ds(base, _CHUNK)], src_v)
        pltpu.sync_copy(dst_hbm.at[pl.ds(base, _CHUNK)], dst_v)
        pltpu.async_copy(hp_hbm.at[src_v], rows_v, sem).wait()
        pltpu.sync_copy(rows_v, acc_sh.at[dst_v], add=True)

    tbase = base0 + _NFULL * _CHUNK
    pltpu.sync_copy(src_hbm.at[pl.ds(tbase, _TAIL)], src_t)
    pltpu.sync_copy(dst_hbm.at[pl.ds(tbase, _TAIL)], dst_t)
    pltpu.async_copy(hp_hbm.at[src_t], rows_t, sem).wait()
    pltpu.sync_copy(rows_t, acc_sh.at[dst_t], add=True)

    plsc.subcore_barrier()
    pltpu.sync_copy(acc_sh.at[pl.ds(s * _RPT, _RPT), :],
                    out_hbm.at[c, pl.ds(s * _RPT, _RPT), :])


_PREC = lax.Precision.HIGHEST


def _mm1_body(p0_ref, p1_ref, x_ref, w_ref, oh_ref, od_ref):
    deg = 1.0 + p0_ref[...] + p1_ref[...]
    dinv = lax.rsqrt(deg)
    g = jnp.dot(x_ref[...], w_ref[...],
                preferred_element_type=jnp.float32, precision=_PREC)
    oh_ref[...] = g * dinv
    od_ref[...] = dinv


@jax.jit
def _mm1_call(p0, p1, x, w):
    grid = (_N // _ROWB,)
    return pl.pallas_call(
        _mm1_body,
        grid=grid,
        in_specs=[
            pl.BlockSpec((_ROWB, 1), lambda i: (i, 0)),
            pl.BlockSpec((_ROWB, 1), lambda i: (i, 0)),
            pl.BlockSpec((_ROWB, _H), lambda i: (i, 0)),
            pl.BlockSpec((_H, _H), lambda i: (0, 0)),
        ],
        out_specs=[
            pl.BlockSpec((_ROWB, _H), lambda i: (i, 0)),
            pl.BlockSpec((_ROWB, 1), lambda i: (i, 0)),
        ],
        out_shape=[
            jax.ShapeDtypeStruct((_N, _H), jnp.float32),
            jax.ShapeDtypeStruct((_N, 1), jnp.float32),
        ],
    )(p0, p1, x, w)


def _layer_body(s0_ref, s1_ref, hp_ref, d_ref, b_ref, w_ref, o_ref):
    y = d_ref[...] * (s0_ref[...] + s1_ref[...] + hp_ref[...]) + b_ref[...]
    y = jnp.maximum(y, 0.0)
    o_ref[...] = jnp.dot(y, w_ref[...],
                         preferred_element_type=jnp.float32,
                         precision=_PREC) * d_ref[...]


@jax.jit
def _layer_call(s0, s1, hp, dinv, b, w):
    grid = (_N // _ROWB,)
    return pl.pallas_call(
        _layer_body,
        grid=grid,
        in_specs=[
            pl.BlockSpec((_ROWB, _H), lambda i: (i, 0)),
            pl.BlockSpec((_ROWB, _H), lambda i: (i, 0)),
            pl.BlockSpec((_ROWB, _H), lambda i: (i, 0)),
            pl.BlockSpec((_ROWB, 1), lambda i: (i, 0)),
            pl.BlockSpec((1, _H), lambda i: (0, 0)),
            pl.BlockSpec((_H, _H), lambda i: (0, 0)),
        ],
        out_specs=pl.BlockSpec((_ROWB, _H), lambda i: (i, 0)),
        out_shape=jax.ShapeDtypeStruct((_N, _H), jnp.float32),
    )(s0, s1, hp, dinv, b, w)


def _final_body(s0_ref, s1_ref, hp_ref, d_ref, b_ref, wl_ref, bl_ref, o_ref):
    y = d_ref[...] * (s0_ref[...] + s1_ref[...] + hp_ref[...]) + b_ref[...]
    y = jnp.maximum(y, 0.0)
    o_ref[...] = jnp.dot(y, wl_ref[...],
                         preferred_element_type=jnp.float32,
                         precision=_PREC) + bl_ref[...]


@jax.jit
def _final_call(s0, s1, hp, dinv, b, wl, bl):
    grid = (_N // _ROWB,)
    return pl.pallas_call(
        _final_body,
        grid=grid,
        in_specs=[
            pl.BlockSpec((_ROWB, _H), lambda i: (i, 0)),
            pl.BlockSpec((_ROWB, _H), lambda i: (i, 0)),
            pl.BlockSpec((_ROWB, _H), lambda i: (i, 0)),
            pl.BlockSpec((_ROWB, 1), lambda i: (i, 0)),
            pl.BlockSpec((1, _H), lambda i: (0, 0)),
            pl.BlockSpec((_H, _C), lambda i: (0, 0)),
            pl.BlockSpec((1, _C), lambda i: (0, 0)),
        ],
        out_specs=pl.BlockSpec((_ROWB, _C), lambda i: (i, 0)),
        out_shape=jax.ShapeDtypeStruct((_N, _C), jnp.float32),
    )(s0, s1, hp, dinv, b, wl, bl)


@jax.jit
def kernel(x, edge_index, W1, b1, W2, b2, W3, b3, Wl, bl):
    src = edge_index[0]
    dst = edge_index[1]
    degp = _deg_call(dst)
    p0 = degp[0, :_N, :1]
    p1 = degp[1, :_N, :1]
    h1, dinv = _mm1_call(p0, p1, x, W1)
    s = _agg_call(h1, src, dst)
    h2 = _layer_call(s[0, :_N], s[1, :_N], h1, dinv, b1.reshape(1, _H), W2)
    s = _agg_call(h2, src, dst)
    h3 = _layer_call(s[0, :_N], s[1, :_N], h2, dinv, b2.reshape(1, _H), W3)
    s = _agg_call(h3, src, dst)
    out = _final_call(s[0, :_N], s[1, :_N], h3, dinv, b3.reshape(1, _H), Wl,
                      bl.reshape(1, _C))
    return out
